# baseline (speedup 1.0000x reference)
import functools

import jax
import jax.numpy as jnp
from jax import lax
from jax.experimental import pallas as pl
from jax.experimental.pallas import tpu as pltpu
from jax.experimental.pallas import tpu_sc as plsc

EMB = 32
R_MASK = 65535
NQ = 15625
NR = 65536
VOCAB = 1000000

QBLK = 8192
QSEL = QBLK // 64
NQBLK = -(-VOCAB // QBLK)
NQ_PAD = NQBLK * QSEL

_info = plsc.get_sparse_core_info()
_NC, _NS, _L = _info.num_cores, _info.num_subcores, _info.num_lanes
_NW = _NC * _NS

CHUNK = 512
SUB = 128
NSUB = CHUNK // SUB


def _eye(n):
    r = lax.broadcasted_iota(jnp.int32, (n, n), 0)
    c = lax.broadcasted_iota(jnp.int32, (n, n), 1)
    return jnp.where(r == c, 1.0, 0.0).astype(jnp.float32)


def _qt_body(t_ref, o_ref):
    x = t_ref[...]
    r = lax.broadcasted_iota(jnp.int32, (QBLK, QSEL), 0)
    c = lax.broadcasted_iota(jnp.int32, (QBLK, QSEL), 1)
    sel = jnp.where(r == c * 64, 1.0, 0.0).astype(jnp.float32)
    y = lax.dot_general(x, sel, (((1,), (0,)), ((), ())),
                        preferred_element_type=jnp.float32)
    o_ref[...] = lax.dot_general(y, _eye(EMB), (((0,), (0,)), ((), ())),
                                 preferred_element_type=jnp.float32)


def _rt_body(t_ref, o_ref):
    x = t_ref[...]
    o_ref[...] = lax.dot_general(x, _eye(EMB), (((0,), (0,)), ((), ())),
                                 preferred_element_type=jnp.float32)


@functools.cache
def _make_distill():
    qt = pl.pallas_call(
        _qt_body,
        grid=(NQBLK,),
        in_specs=[pl.BlockSpec((EMB, QBLK), lambda g: (0, g))],
        out_specs=pl.BlockSpec((QSEL, EMB), lambda g: (g, 0)),
        out_shape=jax.ShapeDtypeStruct((NQ_PAD, EMB), jnp.float32),
    )
    rt = pl.pallas_call(
        _rt_body,
        grid=(NR // QBLK,),
        in_specs=[pl.BlockSpec((EMB, QBLK), lambda g: (0, g))],
        out_specs=pl.BlockSpec((QBLK, EMB), lambda g: (g, 0)),
        out_shape=jax.ShapeDtypeStruct((NR, EMB), jnp.float32),
    )
    return qt, rt


@functools.cache
def _make_lookup(B):
    b_per_w = B // _NW
    n_chunks = b_per_w // CHUNK
    n_pairs = n_chunks // 2
    mesh = plsc.VectorSubcoreMesh(core_axis_name="c", subcore_axis_name="s")

    @functools.partial(
        pl.kernel,
        mesh=mesh,
        out_type=jax.ShapeDtypeStruct((B, EMB), jnp.float32),
        compiler_params=pltpu.CompilerParams(use_tc_tiling_on_sc=False),
        scratch_types=[
            pltpu.VMEM((CHUNK,), jnp.int32),
            pltpu.VMEM((CHUNK,), jnp.int32),
            pltpu.VMEM((CHUNK,), jnp.int32),
            pltpu.VMEM((CHUNK, EMB), jnp.float32),
            pltpu.VMEM((CHUNK,), jnp.int32),
            pltpu.VMEM((CHUNK,), jnp.int32),
            pltpu.VMEM((CHUNK,), jnp.int32),
            pltpu.VMEM((CHUNK, EMB), jnp.float32),
            pltpu.SemaphoreType.DMA,
            pltpu.SemaphoreType.DMA,
            pltpu.SemaphoreType.DMA,
            pltpu.SemaphoreType.DMA,
        ],
    )
    def k(ids_hbm, qt_hbm, rt_hbm, out_hbm,
          ids0, idxq0, idxr0, rq0,
          ids1, idxq1, idxr1, rq1,
          semg0, semg1, semw0, semw1):
        wid = lax.axis_index("s") * _NC + lax.axis_index("c")
        base_w = wid * b_per_w
        slots = ((ids0, idxq0, idxr0, rq0, semg0, semw0),
                 (ids1, idxq1, idxr1, rq1, semg1, semw1))

        def fire_q(s, c, first):
            ids_v, idxq, idxr, rq, semg, semw = slots[s]
            base = base_w + c * CHUNK

            def drain_write():
                pltpu.make_async_copy(
                    rq, out_hbm.at[pl.ds(base, CHUNK)], semw).wait()

            if not first:
                pl.when(c >= 2)(drain_write)

            pltpu.sync_copy(ids_hbm.at[pl.ds(base, CHUNK)], ids_v)

            def idx_body(i, carry):
                v = ids_v[pl.ds(i * _L, _L)]
                hi = lax.shift_right_arithmetic(v, 16)
                j = lax.rem(hi * 1024, jnp.int32(NQ))
                j = jnp.where(j < 0, j + jnp.int32(NQ), j)
                idxq[pl.ds(i * _L, _L)] = j
                idxr[pl.ds(i * _L, _L)] = v & jnp.int32(R_MASK)
                return carry

            lax.fori_loop(0, CHUNK // _L, idx_body, 0)
            for j in range(NSUB):
                pltpu.async_copy(
                    qt_hbm.at[idxq.at[pl.ds(j * SUB, SUB)]],
                    rq.at[pl.ds(j * SUB, SUB)], semg)

        def fire_r(s):
            ids_v, idxq, idxr, rq, semg, semw = slots[s]
            for j in range(NSUB):
                pltpu.make_async_copy(
                    qt_hbm.at[idxq.at[pl.ds(j * SUB, SUB)]],
                    rq.at[pl.ds(j * SUB, SUB)], semg).wait()
            for j in range(NSUB):
                pltpu.async_copy(
                    rt_hbm.at[idxr.at[pl.ds(j * SUB, SUB)]],
                    rq.at[pl.ds(j * SUB, SUB)], semg, add=True)

        def write_out(s, c):
            ids_v, idxq, idxr, rq, semg, semw = slots[s]
            base = base_w + c * CHUNK
            for j in range(NSUB):
                pltpu.make_async_copy(
                    rt_hbm.at[idxr.at[pl.ds(j * SUB, SUB)]],
                    rq.at[pl.ds(j * SUB, SUB)], semg).wait()
            pltpu.async_copy(rq, out_hbm.at[pl.ds(base, CHUNK)], semw)

        fire_q(0, 0, first=True)

        def body(c2, carry):
            c0 = c2 * 2
            c1 = c0 + 1
            fire_q(1, c1, first=False)
            fire_r(0)
            write_out(0, c0)

            def refill():
                fire_q(0, c0 + 2, first=False)

            pl.when(c2 < n_pairs - 1)(refill)
            fire_r(1)
            write_out(1, c1)
            return carry

        lax.fori_loop(0, n_pairs, body, 0)

        last0 = base_w + (n_chunks - 2) * CHUNK
        last1 = base_w + (n_chunks - 1) * CHUNK
        pltpu.make_async_copy(
            rq0, out_hbm.at[pl.ds(last0, CHUNK)], semw0).wait()
        pltpu.make_async_copy(
            rq1, out_hbm.at[pl.ds(last1, CHUNK)], semw1).wait()

    return k


def kernel(ids, table):
    B = ids.shape[0] * ids.shape[1]
    qt_call, rt_call = _make_distill()
    tT = table.T
    qt = qt_call(tT)
    rt = rt_call(tT)
    out = _make_lookup(B)(ids.reshape(-1), qt, rt)
    return out.reshape(ids.shape + (EMB,))
